# baseline (speedup 1.0000x reference)
import functools

import jax
import jax.numpy as jnp
from jax import lax
from jax.experimental import pallas as pl
from jax.experimental.pallas import tpu as pltpu
from jax.experimental.pallas import tpu_sc as plsc

_NC = 2
_NS = 16
_NW = _NC * _NS
_WIN = 200
_FETCH = 208
_AUX = 512
_NBUF = 4
_LAG = 2
_GRP = 16


def _sc_window_gather(idx32, shift16, data, aux):
    B = idx32.shape[0]
    C, D = data.shape
    per_w = B // _NW

    mesh = plsc.VectorSubcoreMesh(
        core_axis_name="c", subcore_axis_name="s",
        num_cores=_NC, num_subcores=_NS)

    @functools.partial(
        pl.kernel,
        out_type=jax.ShapeDtypeStruct((B * _WIN, D), jnp.float32),
        mesh=mesh,
        scratch_types=[
            pltpu.VMEM((per_w,), jnp.int32),
            pltpu.VMEM((16,), jnp.int32),
            pltpu.SMEM((_NBUF,), jnp.int32),
            [pltpu.VMEM((_FETCH, D), jnp.float32) for _ in range(_NBUF)],
            [pltpu.SemaphoreType.DMA for _ in range(2 * _NBUF)],
        ],
    )
    def run(idx_hbm, shift_hbm, tab_hbm, aux_hbm, out_hbm, idx_v, shift_v,
            r0_s, bufs, sems):
        wid = lax.axis_index("c") * _NS + lax.axis_index("s")
        base = wid * per_w
        pltpu.sync_copy(idx_hbm.at[pl.ds(base, per_w)], idx_v)
        pltpu.sync_copy(shift_hbm, shift_v)
        shift = shift_v[pl.ds(0, 16)][0]

        def wait_in(j):
            pltpu.make_async_copy(
                tab_hbm.at[pl.ds(0, _FETCH)], bufs[j], sems[j]).wait()

        def start_out(f, j):
            pltpu.make_async_copy(
                bufs[j].at[pl.ds(r0_s[j], _WIN)],
                out_hbm.at[pl.ds((base + f) * _WIN, _WIN)],
                sems[_NBUF + j]).start()

        def wait_out(j):
            pltpu.make_async_copy(
                bufs[j].at[pl.ds(0, _WIN)], out_hbm.at[pl.ds(0, _WIN)],
                sems[_NBUF + j]).wait()

        def group_body(g, carry):
            v16 = idx_v[pl.ds(g * _GRP, _GRP)]
            for l in range(_GRP):
                e = g * _GRP + l
                j = l % _NBUF

                if l >= _NBUF:
                    wait_out(j)
                else:

                    @pl.when(g > 0)
                    def _():
                        wait_out(j)

                s = v16[l] + shift
                s = jnp.where(s >= C, s - C, s)
                r0 = jnp.bitwise_and(s, 7)
                r0_s[j] = r0
                a = s - r0
                near_end = s > C - _FETCH

                @pl.when(near_end)
                def _():
                    pltpu.make_async_copy(
                        aux_hbm.at[pl.ds(
                            pl.multiple_of(a - (C - _AUX // 2), 8), _FETCH)],
                        bufs[j], sems[j]).start()

                @pl.when(jnp.logical_not(near_end))
                def _():
                    pltpu.make_async_copy(
                        tab_hbm.at[pl.ds(pl.multiple_of(a, 8), _FETCH)],
                        bufs[j], sems[j]).start()

                f = e - _LAG
                fj = (l - _LAG) % _NBUF
                if l >= _LAG:
                    wait_in(fj)
                    start_out(f, fj)
                else:

                    @pl.when(g > 0)
                    def _():
                        wait_in(fj)
                        start_out(f, fj)
            return carry

        lax.fori_loop(0, per_w // _GRP, group_body, jnp.int32(0))

        for r in range(_LAG):
            f = per_w - _LAG + r
            fj = f % _NBUF
            wait_in(fj)
            start_out(jnp.int32(f), fj)
        for j in range(_NBUF):
            wait_out(j)

    return run(idx32, shift16, data, aux)


def _tc_window_gather(starts, data, aux):
    B2 = starts.shape[0]
    C, D = data.shape
    nb, lag = 4, 2

    def body(sref, dref, aref, oref, buf, r0_s, sin, sout):
        b = pl.program_id(0)
        slot = lax.rem(b, nb)

        @pl.when(b >= nb)
        def _():
            pltpu.make_async_copy(
                buf.at[slot, pl.ds(0, _WIN)], oref.at[0],
                sout.at[slot]).wait()

        s = sref[b]
        r0 = jnp.bitwise_and(s, 7)
        r0_s[slot] = r0
        a = s - r0
        near_end = s > C - _FETCH

        @pl.when(near_end)
        def _():
            pltpu.make_async_copy(
                aref.at[pl.ds(pl.multiple_of(a - (C - _AUX // 2), 8),
                              _FETCH)],
                buf.at[slot], sin.at[slot]).start()

        @pl.when(jnp.logical_not(near_end))
        def _():
            pltpu.make_async_copy(
                dref.at[pl.ds(pl.multiple_of(a, 8), _FETCH)],
                buf.at[slot], sin.at[slot]).start()

        def fire_out(f):
            fs = lax.rem(f, nb)
            pltpu.make_async_copy(
                dref.at[pl.ds(0, _FETCH)], buf.at[fs], sin.at[fs]).wait()
            pltpu.make_async_copy(
                buf.at[fs, pl.ds(r0_s[fs], _WIN)], oref.at[f],
                sout.at[fs]).start()

        @pl.when(b >= lag)
        def _():
            fire_out(b - lag)

        @pl.when(b == B2 - 1)
        def _():
            for r in range(lag):
                fire_out(jnp.int32(B2 - lag + r))
            for j in range(nb):
                pltpu.make_async_copy(
                    buf.at[j, pl.ds(0, _WIN)], oref.at[0],
                    sout.at[j]).wait()

    grid_spec = pltpu.PrefetchScalarGridSpec(
        num_scalar_prefetch=1,
        grid=(B2,),
        in_specs=[pl.BlockSpec(memory_space=pl.ANY),
                  pl.BlockSpec(memory_space=pl.ANY)],
        out_specs=pl.BlockSpec(memory_space=pl.ANY),
        scratch_shapes=[
            pltpu.VMEM((nb, _FETCH, D), jnp.float32),
            pltpu.SMEM((nb,), jnp.int32),
            pltpu.SemaphoreType.DMA((nb,)),
            pltpu.SemaphoreType.DMA((nb,)),
        ],
    )
    return pl.pallas_call(
        body,
        grid_spec=grid_spec,
        out_shape=jax.ShapeDtypeStruct((B2, _WIN, D), jnp.float32),
        compiler_params=pltpu.CompilerParams(
            dimension_semantics=("arbitrary",)),
    )(starts, data, aux)


def kernel(index, length, data):
    C, D = data.shape
    B = index.shape[0]
    B1 = B // 2
    idx32 = index.astype(jnp.int32)
    shift = jnp.mod(jnp.asarray(length, jnp.int32) - _WIN, C)
    shift16 = jnp.full((16,), shift, jnp.int32)
    aux = jnp.concatenate([data[C - _AUX // 2:], data[:_AUX // 2]], axis=0)
    sc_out = _sc_window_gather(idx32[:B1], shift16, data, aux)
    starts_tc = jnp.mod(idx32[B1:] + shift, C)
    tc_out = _tc_window_gather(starts_tc, data, aux)
    return jnp.concatenate(
        [sc_out.reshape(B1, _WIN, D), tc_out], axis=0)


# speedup vs baseline: 2.5921x; 2.5921x over previous
import functools

import jax
import jax.numpy as jnp
from jax import lax
from jax.experimental import pallas as pl
from jax.experimental.pallas import tpu as pltpu
from jax.experimental.pallas import tpu_sc as plsc

_NC = 2
_NS = 16
_NW = _NC * _NS
_WIN = 200
_CH0 = 104
_CH1 = _WIN - _CH0
_FETCH = 112
_AUX = 512
_NBUF = 8
_LAG = 4
_GRP = 16


def _sc_window_gather(idx32, shift16, data, aux):
    B = idx32.shape[0]
    C, D = data.shape
    per_w = B // _NW

    mesh = plsc.VectorSubcoreMesh(
        core_axis_name="c", subcore_axis_name="s",
        num_cores=_NC, num_subcores=_NS)

    @functools.partial(
        pl.kernel,
        out_type=jax.ShapeDtypeStruct((B * _WIN, D), jnp.float32),
        mesh=mesh,
        scratch_types=[
            pltpu.VMEM((per_w,), jnp.int32),
            pltpu.VMEM((16,), jnp.int32),
            pltpu.SMEM((_NBUF,), jnp.int32),
            [pltpu.VMEM((_FETCH, D), jnp.float32) for _ in range(_NBUF)],
            [pltpu.SemaphoreType.DMA for _ in range(2 * _NBUF)],
        ],
    )
    def run(idx_hbm, shift_hbm, tab_hbm, aux_hbm, out_hbm, idx_v, shift_v,
            r0_s, bufs, sems):
        wid = lax.axis_index("c") * _NS + lax.axis_index("s")
        base = wid * per_w
        pltpu.sync_copy(idx_hbm.at[pl.ds(base, per_w)], idx_v)
        pltpu.sync_copy(shift_hbm, shift_v)
        shift = shift_v[pl.ds(0, 16)][0]

        def chlen(j):
            return _CH0 if j % 2 == 0 else _CH1

        def wait_in(j):
            pltpu.make_async_copy(
                tab_hbm.at[pl.ds(0, _FETCH)], bufs[j], sems[j]).wait()

        def start_out(e, h, j):
            pltpu.make_async_copy(
                bufs[j].at[pl.ds(r0_s[j], chlen(j))],
                out_hbm.at[pl.ds((base + e) * _WIN + _CH0 * h, chlen(j))],
                sems[_NBUF + j]).start()

        def wait_out(j):
            pltpu.make_async_copy(
                bufs[j].at[pl.ds(0, chlen(j))],
                out_hbm.at[pl.ds(0, chlen(j))],
                sems[_NBUF + j]).wait()

        def start_in(s_c, j):
            r0 = jnp.bitwise_and(s_c, 7)
            r0_s[j] = r0
            a = s_c - r0
            near_end = s_c > C - _FETCH

            @pl.when(near_end)
            def _():
                pltpu.make_async_copy(
                    aux_hbm.at[pl.ds(
                        pl.multiple_of(a - (C - _AUX // 2), 8), _FETCH)],
                    bufs[j], sems[j]).start()

            @pl.when(jnp.logical_not(near_end))
            def _():
                pltpu.make_async_copy(
                    tab_hbm.at[pl.ds(pl.multiple_of(a, 8), _FETCH)],
                    bufs[j], sems[j]).start()

        def group_body(g, carry):
            v16 = idx_v[pl.ds(g * _GRP, _GRP)]
            for l in range(_GRP):
                s = v16[l] + shift
                s = jnp.where(s >= C, s - C, s)
                for h in range(2):
                    cl = 2 * l + h
                    j = cl % _NBUF

                    if cl >= _NBUF:
                        wait_out(j)
                    else:

                        @pl.when(g > 0)
                        def _():
                            wait_out(j)

                    if h == 0:
                        s_c = s
                    else:
                        s_c = s + _CH0
                        s_c = jnp.where(s_c >= C, s_c - C, s_c)
                    start_in(s_c, j)

                    fl = cl - _LAG
                    fj = fl % _NBUF
                    fh = fl % 2
                    fe_stat = (fl - fh) // 2
                    if fl >= 0:
                        wait_in(fj)
                        start_out(g * _GRP + fe_stat, fh, fj)
                    else:

                        @pl.when(g > 0)
                        def _():
                            wait_in(fj)
                            start_out(g * _GRP + fe_stat, fh, fj)
            return carry

        lax.fori_loop(0, per_w // _GRP, group_body, jnp.int32(0))

        nch = 2 * per_w
        for r in range(_LAG):
            fl = nch - _LAG + r
            fj = fl % _NBUF
            fh = fl % 2
            fe = (fl - fh) // 2
            wait_in(fj)
            start_out(jnp.int32(fe), fh, fj)
        for j in range(_NBUF):
            wait_out(j)

    return run(idx32, shift16, data, aux)


def kernel(index, length, data):
    C, D = data.shape
    B = index.shape[0]
    idx32 = index.astype(jnp.int32)
    shift = jnp.mod(jnp.asarray(length, jnp.int32) - _WIN, C)
    shift16 = jnp.full((16,), shift, jnp.int32)
    aux = jnp.concatenate([data[C - _AUX // 2:], data[:_AUX // 2]], axis=0)
    out = _sc_window_gather(idx32, shift16, data, aux)
    return out.reshape(B, _WIN, D)


# speedup vs baseline: 2.6229x; 1.0119x over previous
import functools

import jax
import jax.numpy as jnp
from jax import lax
from jax.experimental import pallas as pl
from jax.experimental.pallas import tpu as pltpu
from jax.experimental.pallas import tpu_sc as plsc

_NC = 2
_NS = 16
_NW = _NC * _NS
_WIN = 200
_FETCH = 208
_AUX = 512
_NBUF = 4
_LAG = 2
_GRP = 16


def _sc_window_gather(idx32, shift16, data, aux):
    B = idx32.shape[0]
    C, D = data.shape
    per_w = B // _NW

    mesh = plsc.VectorSubcoreMesh(
        core_axis_name="c", subcore_axis_name="s",
        num_cores=_NC, num_subcores=_NS)

    @functools.partial(
        pl.kernel,
        out_type=jax.ShapeDtypeStruct((B * _WIN, D), jnp.float32),
        mesh=mesh,
        scratch_types=[
            pltpu.VMEM((per_w,), jnp.int32),
            pltpu.VMEM((16,), jnp.int32),
            pltpu.SMEM((_NBUF,), jnp.int32),
            [pltpu.VMEM((_FETCH, D), jnp.float32) for _ in range(_NBUF)],
            [pltpu.SemaphoreType.DMA for _ in range(2 * _NBUF)],
        ],
    )
    def run(idx_hbm, shift_hbm, tab_hbm, aux_hbm, out_hbm, idx_v, shift_v,
            r0_s, bufs, sems):
        wid = lax.axis_index("c") * _NS + lax.axis_index("s")
        base = wid * per_w
        pltpu.sync_copy(idx_hbm.at[pl.ds(base, per_w)], idx_v)
        pltpu.sync_copy(shift_hbm, shift_v)
        shift = shift_v[pl.ds(0, 16)][0]

        def wait_in(j):
            pltpu.make_async_copy(
                tab_hbm.at[pl.ds(0, _FETCH)], bufs[j], sems[j]).wait()

        def start_out(f, j):
            pltpu.make_async_copy(
                bufs[j].at[pl.ds(r0_s[j], _WIN)],
                out_hbm.at[pl.ds((base + f) * _WIN, _WIN)],
                sems[_NBUF + j]).start()

        def wait_out(j):
            pltpu.make_async_copy(
                bufs[j].at[pl.ds(0, _WIN)], out_hbm.at[pl.ds(0, _WIN)],
                sems[_NBUF + j]).wait()

        def group_body(g, carry):
            v16 = idx_v[pl.ds(g * _GRP, _GRP)]
            for l in range(_GRP):
                e = g * _GRP + l
                j = l % _NBUF

                if l >= _NBUF:
                    wait_out(j)
                else:

                    @pl.when(g > 0)
                    def _():
                        wait_out(j)

                s = v16[l] + shift
                s = jnp.where(s >= C, s - C, s)
                r0 = jnp.bitwise_and(s, 7)
                r0_s[j] = r0
                a = s - r0
                near_end = s > C - _FETCH

                @pl.when(near_end)
                def _():
                    pltpu.make_async_copy(
                        aux_hbm.at[pl.ds(
                            pl.multiple_of(a - (C - _AUX // 2), 8), _FETCH)],
                        bufs[j], sems[j]).start()

                @pl.when(jnp.logical_not(near_end))
                def _():
                    pltpu.make_async_copy(
                        tab_hbm.at[pl.ds(pl.multiple_of(a, 8), _FETCH)],
                        bufs[j], sems[j]).start()

                f = e - _LAG
                fj = (l - _LAG) % _NBUF
                if l >= _LAG:
                    wait_in(fj)
                    start_out(f, fj)
                else:

                    @pl.when(g > 0)
                    def _():
                        wait_in(fj)
                        start_out(f, fj)
            return carry

        lax.fori_loop(0, per_w // _GRP, group_body, jnp.int32(0))

        for r in range(_LAG):
            f = per_w - _LAG + r
            fj = f % _NBUF
            wait_in(fj)
            start_out(jnp.int32(f), fj)
        for j in range(_NBUF):
            wait_out(j)

    return run(idx32, shift16, data, aux)


def kernel(index, length, data):
    C, D = data.shape
    B = index.shape[0]
    idx32 = index.astype(jnp.int32)
    shift = jnp.mod(jnp.asarray(length, jnp.int32) - _WIN, C)
    shift16 = jnp.full((16,), shift, jnp.int32)
    aux = jnp.concatenate([data[C - _AUX // 2:], data[:_AUX // 2]], axis=0)
    out = _sc_window_gather(idx32, shift16, data, aux)
    return out.reshape(B, _WIN, D)


# speedup vs baseline: 2.6257x; 1.0011x over previous
import functools

import jax
import jax.numpy as jnp
from jax import lax
from jax.experimental import pallas as pl
from jax.experimental.pallas import tpu as pltpu
from jax.experimental.pallas import tpu_sc as plsc

_NC = 2
_NS = 16
_NW = _NC * _NS
_WIN = 200
_FETCH = 208
_AUX = 512
_NBUF = 4
_LAG = 1
_GRP = 16


def _sc_window_gather(idx32, shift16, data, aux):
    B = idx32.shape[0]
    C, D = data.shape
    per_w = B // _NW

    mesh = plsc.VectorSubcoreMesh(
        core_axis_name="c", subcore_axis_name="s",
        num_cores=_NC, num_subcores=_NS)

    @functools.partial(
        pl.kernel,
        out_type=jax.ShapeDtypeStruct((B * _WIN, D), jnp.float32),
        mesh=mesh,
        scratch_types=[
            pltpu.VMEM((per_w,), jnp.int32),
            pltpu.VMEM((16,), jnp.int32),
            pltpu.SMEM((_NBUF,), jnp.int32),
            [pltpu.VMEM((_FETCH, D), jnp.float32) for _ in range(_NBUF)],
            [pltpu.SemaphoreType.DMA for _ in range(2 * _NBUF)],
        ],
    )
    def run(idx_hbm, shift_hbm, tab_hbm, aux_hbm, out_hbm, idx_v, shift_v,
            r0_s, bufs, sems):
        wid = lax.axis_index("c") * _NS + lax.axis_index("s")
        base = wid * per_w
        pltpu.sync_copy(idx_hbm.at[pl.ds(base, per_w)], idx_v)
        pltpu.sync_copy(shift_hbm, shift_v)
        shift = shift_v[pl.ds(0, 16)][0]

        def wait_in(j):
            pltpu.make_async_copy(
                tab_hbm.at[pl.ds(0, _FETCH)], bufs[j], sems[j]).wait()

        def start_out(f, j):
            pltpu.make_async_copy(
                bufs[j].at[pl.ds(r0_s[j], _WIN)],
                out_hbm.at[pl.ds((base + f) * _WIN, _WIN)],
                sems[_NBUF + j]).start()

        def wait_out(j):
            pltpu.make_async_copy(
                bufs[j].at[pl.ds(0, _WIN)], out_hbm.at[pl.ds(0, _WIN)],
                sems[_NBUF + j]).wait()

        def group_body(g, carry):
            v16 = idx_v[pl.ds(g * _GRP, _GRP)]
            for l in range(_GRP):
                e = g * _GRP + l
                j = l % _NBUF

                if l >= _NBUF:
                    wait_out(j)
                else:

                    @pl.when(g > 0)
                    def _():
                        wait_out(j)

                s = v16[l] + shift
                s = jnp.where(s >= C, s - C, s)
                r0 = jnp.bitwise_and(s, 7)
                r0_s[j] = r0
                a = s - r0
                near_end = s > C - _FETCH

                @pl.when(near_end)
                def _():
                    pltpu.make_async_copy(
                        aux_hbm.at[pl.ds(
                            pl.multiple_of(a - (C - _AUX // 2), 8), _FETCH)],
                        bufs[j], sems[j]).start()

                @pl.when(jnp.logical_not(near_end))
                def _():
                    pltpu.make_async_copy(
                        tab_hbm.at[pl.ds(pl.multiple_of(a, 8), _FETCH)],
                        bufs[j], sems[j]).start()

                f = e - _LAG
                fj = (l - _LAG) % _NBUF
                if l >= _LAG:
                    wait_in(fj)
                    start_out(f, fj)
                else:

                    @pl.when(g > 0)
                    def _():
                        wait_in(fj)
                        start_out(f, fj)
            return carry

        lax.fori_loop(0, per_w // _GRP, group_body, jnp.int32(0))

        for r in range(_LAG):
            f = per_w - _LAG + r
            fj = f % _NBUF
            wait_in(fj)
            start_out(jnp.int32(f), fj)
        for j in range(_NBUF):
            wait_out(j)

    return run(idx32, shift16, data, aux)


def kernel(index, length, data):
    C, D = data.shape
    B = index.shape[0]
    idx32 = index.astype(jnp.int32)
    shift = jnp.mod(jnp.asarray(length, jnp.int32) - _WIN, C)
    shift16 = jnp.full((16,), shift, jnp.int32)
    aux = jnp.concatenate([data[C - _AUX // 2:], data[:_AUX // 2]], axis=0)
    out = _sc_window_gather(idx32, shift16, data, aux)
    return out.reshape(B, _WIN, D)
